# baseline (speedup 1.0000x reference)
import functools

import jax
import jax.numpy as jnp
from jax import lax
from jax.experimental import pallas as pl
from jax.experimental.pallas import tpu as pltpu

N_NODES = 10000
N_EDGES = 320000
NUM_ENC_STEPS = 12
NUM_CLASS_STEPS = 11

E_PAD = 323584
BE = 4096


def _relu(v):
    return jnp.maximum(v, 0.0)


def _node_enc_body(x_ref, w1_ref, b1_ref, w2_ref, b2_ref, x0_ref, xin_ref):
    h = _relu(jnp.dot(x_ref[...], w1_ref[...],
                      preferred_element_type=jnp.float32) + b1_ref[...])
    x0 = _relu(jnp.dot(h, w2_ref[...],
                       preferred_element_type=jnp.float32) + b2_ref[...])
    x0_ref[...] = x0
    xin_ref[:, 0:32] = x0
    xin_ref[:, 32:64] = x0


def _node_enc(x, w1, b1, w2, b2):
    return pl.pallas_call(
        _node_enc_body,
        out_shape=(jax.ShapeDtypeStruct((N_NODES, 32), jnp.float32),
                   jax.ShapeDtypeStruct((N_NODES, 64), jnp.float32)),
    )(x, w1, b1, w2, b2)


def _edge_enc_body(a_ref, w1_ref, b1_ref, w2_ref, b2_ref, e0_ref):
    h = _relu(jnp.dot(a_ref[...], w1_ref[...],
                      preferred_element_type=jnp.float32) + b1_ref[...])
    e0_ref[...] = _relu(jnp.dot(h, w2_ref[...],
                                preferred_element_type=jnp.float32) + b2_ref[...])


def _edge_enc(attr, w1, b1, w2, b2):
    nb = E_PAD // BE
    return pl.pallas_call(
        _edge_enc_body,
        grid=(nb,),
        in_specs=[
            pl.BlockSpec((BE, 4), lambda i: (i, 0)),
            pl.BlockSpec((4, 16), lambda i: (0, 0)),
            pl.BlockSpec((1, 16), lambda i: (0, 0)),
            pl.BlockSpec((16, 16), lambda i: (0, 0)),
            pl.BlockSpec((1, 16), lambda i: (0, 0)),
        ],
        out_specs=pl.BlockSpec((BE, 16), lambda i: (i, 0)),
        out_shape=jax.ShapeDtypeStruct((E_PAD, 16), jnp.float32),
    )(attr, w1, b1, w2, b2)


def _edge_step_body(xr_ref, xc_ref, e0_ref, el_ref,
                    w1a_ref, w1b_ref, w1c_ref, w1d_ref, b1_ref,
                    w2_ref, b2_ref,
                    m1x_ref, m1e_ref, mb1_ref, m2_ref, mb2_ref,
                    c1_ref, cb1_ref, c2_ref, cb2_ref,
                    el_out_ref, mcol_ref, mrow_ref, logit_ref):
    xr = xr_ref[...]
    xc = xc_ref[...]
    f32 = jnp.float32
    h = (jnp.dot(xr, w1a_ref[...], preferred_element_type=f32)
         + jnp.dot(xc, w1b_ref[...], preferred_element_type=f32)
         + jnp.dot(e0_ref[...], w1c_ref[...], preferred_element_type=f32)
         + jnp.dot(el_ref[...], w1d_ref[...], preferred_element_type=f32)
         + b1_ref[...])
    h = _relu(h)
    el_new = _relu(jnp.dot(h, w2_ref[...], preferred_element_type=f32)
                   + b2_ref[...])
    el_out_ref[...] = el_new
    ecomp = jnp.dot(el_new, m1e_ref[...], preferred_element_type=f32) + mb1_ref[...]
    hc = _relu(jnp.dot(xr, m1x_ref[...], preferred_element_type=f32) + ecomp)
    hr = _relu(jnp.dot(xc, m1x_ref[...], preferred_element_type=f32) + ecomp)
    mcol_ref[...] = _relu(jnp.dot(hc, m2_ref[...], preferred_element_type=f32)
                          + mb2_ref[...])
    mrow_ref[...] = _relu(jnp.dot(hr, m2_ref[...], preferred_element_type=f32)
                          + mb2_ref[...])
    g = _relu(jnp.dot(el_new, c1_ref[...], preferred_element_type=f32)
              + cb1_ref[...])
    logit_ref[...] = (jnp.dot(g, c2_ref[...], preferred_element_type=f32)
                      + cb2_ref[...])


def _edge_step(xr, xc, e0, el, ew):
    nb = E_PAD // BE
    wspec = lambda r, c: pl.BlockSpec((r, c), lambda i: (0, 0))
    return pl.pallas_call(
        _edge_step_body,
        grid=(nb,),
        in_specs=[
            pl.BlockSpec((BE, 64), lambda i: (i, 0)),
            pl.BlockSpec((BE, 64), lambda i: (i, 0)),
            pl.BlockSpec((BE, 16), lambda i: (i, 0)),
            pl.BlockSpec((BE, 16), lambda i: (i, 0)),
            wspec(64, 80), wspec(64, 80), wspec(16, 80), wspec(16, 80),
            wspec(1, 80), wspec(80, 16), wspec(1, 16),
            wspec(64, 56), wspec(16, 56), wspec(1, 56),
            wspec(56, 32), wspec(1, 32),
            wspec(16, 8), wspec(1, 8), wspec(8, 1), wspec(1, 1),
        ],
        out_specs=(
            pl.BlockSpec((BE, 16), lambda i: (i, 0)),
            pl.BlockSpec((BE, 32), lambda i: (i, 0)),
            pl.BlockSpec((BE, 32), lambda i: (i, 0)),
            pl.BlockSpec((BE, 1), lambda i: (i, 0)),
        ),
        out_shape=(
            jax.ShapeDtypeStruct((E_PAD, 16), jnp.float32),
            jax.ShapeDtypeStruct((E_PAD, 32), jnp.float32),
            jax.ShapeDtypeStruct((E_PAD, 32), jnp.float32),
            jax.ShapeDtypeStruct((E_PAD, 1), jnp.float32),
        ),
    )(xr, xc, e0, el, *ew)


def _update_body(a0_ref, a1_ref, x0_ref, wu_ref, bu_ref, xin_ref):
    agg = a0_ref[...] + a1_ref[...]
    xl = _relu(jnp.dot(agg, wu_ref[...], preferred_element_type=jnp.float32)
               + bu_ref[...])
    xin_ref[:, 0:32] = x0_ref[...]
    xin_ref[:, 32:64] = xl


def _update(agg0, agg1, x0, wu, bu):
    return pl.pallas_call(
        _update_body,
        out_shape=jax.ShapeDtypeStruct((N_NODES, 64), jnp.float32),
    )(agg0, agg1, x0, wu, bu)


def _gather(xin, idx_g):
    return jnp.take(xin, idx_g, axis=0)


def _scatter_add(m, idx_s):
    z = jnp.zeros((N_NODES + 1, 32), jnp.float32)
    return z.at[idx_s].add(m)[:N_NODES]


def kernel(x, edge_index, edge_attr, params):
    row, col = edge_index[0], edge_index[1]
    pad = E_PAD - N_EDGES
    row_g = jnp.concatenate([row, jnp.zeros((pad,), jnp.int32)])
    col_g = jnp.concatenate([col, jnp.zeros((pad,), jnp.int32)])
    row_s = jnp.concatenate([row, jnp.full((pad,), N_NODES, jnp.int32)])
    col_s = jnp.concatenate([col, jnp.full((pad,), N_NODES, jnp.int32)])
    attr_p = jnp.concatenate(
        [edge_attr, jnp.zeros((pad, 4), jnp.float32)], axis=0)

    (we1, be1), (we2, be2) = params['enc_node']
    (ee1, eb1), (ee2, eb2) = params['enc_edge']
    (w1, b1), (w2, b2) = params['edge_model']
    (m1, mb1), (m2, mb2) = params['msg']
    ((wu, bu),) = params['update']
    (c1, cb1), (c2, cb2) = params['cls_edge']

    row2 = lambda b: b.reshape(1, -1)
    ew = (w1[0:64], w1[64:128], w1[128:144], w1[144:160], row2(b1),
          w2, row2(b2),
          m1[0:64], m1[64:80], row2(mb1), m2, row2(mb2),
          c1, row2(cb1), c2, row2(cb2))

    x0, xin = _node_enc(x, we1, row2(be1), we2, row2(be2))
    e0 = _edge_enc(attr_p, ee1, row2(eb1), ee2, row2(eb2))
    el = e0

    outs = []
    first_class_step = NUM_ENC_STEPS - NUM_CLASS_STEPS + 1
    for step in range(1, NUM_ENC_STEPS + 1):
        xr = _gather(xin, row_g)
        xc = _gather(xin, col_g)
        el, mcol, mrow, logit = _edge_step(xr, xc, e0, el, ew)
        agg0 = _scatter_add(mcol, col_s)
        agg1 = _scatter_add(mrow, row_s)
        xin = _update(agg0, agg1, x0, wu, bu)
        if step >= first_class_step:
            outs.append(logit[:N_EDGES])
    return jnp.stack(outs, axis=0)


# speedup vs baseline: 1.4695x; 1.4695x over previous
import functools

import jax
import jax.numpy as jnp
from jax import lax
from jax.experimental import pallas as pl
from jax.experimental.pallas import tpu as pltpu
from jax.experimental.pallas import tpu_sc as plsc

N_NODES = 10000
N_EDGES = 320000
NUM_ENC_STEPS = 12
NUM_CLASS_STEPS = 11

NC = 2
NS = 16
NW = NC * NS
CH = 128
K = 80
E_PAD = NW * K * CH
BE = 4096
N_PAD = 10240


def _relu(v):
    return jnp.maximum(v, 0.0)


def _node_enc_body(x_ref, w1_ref, b1_ref, w2_ref, b2_ref, x0_ref, xin_ref):
    h = _relu(jnp.dot(x_ref[...], w1_ref[...],
                      preferred_element_type=jnp.float32) + b1_ref[...])
    x0 = _relu(jnp.dot(h, w2_ref[...],
                       preferred_element_type=jnp.float32) + b2_ref[...])
    x0_ref[...] = x0
    xin_ref[:, 0:32] = x0
    xin_ref[:, 32:64] = x0
    xin_ref[:, 64:128] = jnp.zeros((N_NODES, 64), jnp.float32)


def _node_enc(x, w1, b1, w2, b2):
    return pl.pallas_call(
        _node_enc_body,
        out_shape=(jax.ShapeDtypeStruct((N_NODES, 32), jnp.float32),
                   jax.ShapeDtypeStruct((N_NODES, 128), jnp.float32)),
    )(x, w1, b1, w2, b2)


def _edge_enc_body(a_ref, w1_ref, b1_ref, w2_ref, b2_ref, e0_ref):
    h = _relu(jnp.dot(a_ref[...], w1_ref[...],
                      preferred_element_type=jnp.float32) + b1_ref[...])
    e0_ref[...] = _relu(jnp.dot(h, w2_ref[...],
                                preferred_element_type=jnp.float32) + b2_ref[...])


def _edge_enc(attr, w1, b1, w2, b2):
    nb = E_PAD // BE
    return pl.pallas_call(
        _edge_enc_body,
        grid=(nb,),
        in_specs=[
            pl.BlockSpec((BE, 4), lambda i: (i, 0)),
            pl.BlockSpec((4, 16), lambda i: (0, 0)),
            pl.BlockSpec((1, 16), lambda i: (0, 0)),
            pl.BlockSpec((16, 16), lambda i: (0, 0)),
            pl.BlockSpec((1, 16), lambda i: (0, 0)),
        ],
        out_specs=pl.BlockSpec((BE, 16), lambda i: (i, 0)),
        out_shape=jax.ShapeDtypeStruct((E_PAD, 16), jnp.float32),
    )(attr, w1, b1, w2, b2)


def _edge_step_body(xr_ref, xc_ref, e0_ref, el_ref,
                    w1_ref, b1_ref,
                    w2_ref, b2_ref,
                    mc1_ref, mcb_ref, bdx_ref, bdm_ref, bmb2_ref,
                    c2_ref, cb2_ref,
                    el_out_ref, mcol_ref, mrow_ref, logit_ref):
    xr = xr_ref[:, 0:64]
    xc = xc_ref[:, 0:64]
    f32 = jnp.float32
    cat1 = jnp.concatenate([xr, xc, e0_ref[...], el_ref[...]], axis=1)
    h = _relu(jnp.dot(cat1, w1_ref[...], preferred_element_type=f32)
              + b1_ref[...])
    el_new = _relu(jnp.dot(h, w2_ref[...], preferred_element_type=f32)
                   + b2_ref[...])
    el_out_ref[...] = el_new
    d2 = jnp.dot(el_new, mc1_ref[...], preferred_element_type=f32) + mcb_ref[...]
    ecomp2 = jnp.concatenate([d2[:, 0:64], d2[:, 0:64]], axis=1)
    hcr = _relu(jnp.dot(cat1[:, 0:128], bdx_ref[...],
                        preferred_element_type=f32) + ecomp2)
    mcr = _relu(jnp.dot(hcr, bdm_ref[...], preferred_element_type=f32)
                + bmb2_ref[...])
    mcol_ref[...] = mcr[:, 0:32]
    mrow_ref[...] = mcr[:, 32:64]
    g = _relu(d2[:, 64:72])
    logit_ref[...] = (jnp.dot(g, c2_ref[...], preferred_element_type=f32)
                      + cb2_ref[...])


def _edge_step(xr, xc, e0, el, ew):
    nb = E_PAD // BE
    wspec = lambda r, c: pl.BlockSpec((r, c), lambda i: (0, 0))
    return pl.pallas_call(
        _edge_step_body,
        grid=(nb,),
        in_specs=[
            pl.BlockSpec((BE, 128), lambda i: (i, 0)),
            pl.BlockSpec((BE, 128), lambda i: (i, 0)),
            pl.BlockSpec((BE, 16), lambda i: (i, 0)),
            pl.BlockSpec((BE, 16), lambda i: (i, 0)),
            wspec(160, 80),
            wspec(1, 80), wspec(80, 16), wspec(1, 16),
            wspec(16, 72), wspec(1, 72),
            wspec(128, 128), wspec(128, 64), wspec(1, 64),
            wspec(8, 1), wspec(1, 1),
        ],
        out_specs=(
            pl.BlockSpec((BE, 16), lambda i: (i, 0)),
            pl.BlockSpec((BE, 32), lambda i: (i, 0)),
            pl.BlockSpec((BE, 32), lambda i: (i, 0)),
            pl.BlockSpec((BE, 1), lambda i: (i, 0)),
        ),
        out_shape=(
            jax.ShapeDtypeStruct((E_PAD, 16), jnp.float32),
            jax.ShapeDtypeStruct((E_PAD, 32), jnp.float32),
            jax.ShapeDtypeStruct((E_PAD, 32), jnp.float32),
            jax.ShapeDtypeStruct((E_PAD, 1), jnp.float32),
        ),
    )(xr, xc, e0, el, *ew)


def _update_body(agg_ref, x0_ref, wu_ref, bu_ref, xin_ref):
    agg = agg_ref[0, :N_NODES, :] + agg_ref[1, :N_NODES, :]
    xl = _relu(jnp.dot(agg, wu_ref[...], preferred_element_type=jnp.float32)
               + bu_ref[...])
    xin_ref[:, 0:32] = x0_ref[...]
    xin_ref[:, 32:64] = xl
    xin_ref[:, 64:128] = jnp.zeros((N_NODES, 64), jnp.float32)


def _update(agg2, x0, wu, bu):
    return pl.pallas_call(
        _update_body,
        out_shape=jax.ShapeDtypeStruct((N_NODES, 128), jnp.float32),
    )(agg2, x0, wu, bu)


@functools.cache
def _sc_mesh():
    return plsc.VectorSubcoreMesh(core_axis_name="c", subcore_axis_name="s",
                                  num_cores=NC, num_subcores=NS)


def _sc_gather_body(xin_hbm, rowg_hbm, colg_hbm, xr_hbm, xc_hbm,
                    idx_v, buf0, buf1, sem0, sem1):
    c = lax.axis_index("c")
    s = lax.axis_index("s")
    w = s * NC + c
    base = w * (K * CH)
    for idx_hbm, out_hbm in ((rowg_hbm, xr_hbm), (colg_hbm, xc_hbm)):
        pltpu.sync_copy(idx_hbm.at[w], idx_v)

        def step(j, carry):
            pltpu.async_copy(xin_hbm.at[idx_v.at[j]], buf0, sem0).wait()
            pltpu.sync_copy(buf0, out_hbm.at[pl.ds(base + j * CH, CH)])
            return carry

        lax.fori_loop(0, K, step, 0)


def _gather2(xin, rowg3, colg3):
    return pl.kernel(
        _sc_gather_body,
        out_type=(jax.ShapeDtypeStruct((E_PAD, 128), jnp.float32),
                  jax.ShapeDtypeStruct((E_PAD, 128), jnp.float32)),
        mesh=_sc_mesh(),
        scratch_types=[
            pltpu.VMEM((K, CH), jnp.int32),
            pltpu.VMEM((CH, 128), jnp.float32),
            pltpu.VMEM((CH, 128), jnp.float32),
            pltpu.SemaphoreType.DMA,
            pltpu.SemaphoreType.DMA,
        ],
    )(xin, rowg3, colg3)


def kernel(x, edge_index, edge_attr, params):
    row, col = edge_index[0], edge_index[1]
    pad = E_PAD - N_EDGES
    row_g = jnp.concatenate([row, jnp.zeros((pad,), jnp.int32)]).reshape(
        NW, K, CH)
    col_g = jnp.concatenate([col, jnp.zeros((pad,), jnp.int32)]).reshape(
        NW, K, CH)
    row_s = jnp.concatenate([row, jnp.full((pad,), N_NODES, jnp.int32)])
    col_s = jnp.concatenate([col, jnp.full((pad,), N_NODES, jnp.int32)])
    attr_p = jnp.concatenate(
        [edge_attr, jnp.zeros((pad, 4), jnp.float32)], axis=0)

    (we1, be1), (we2, be2) = params['enc_node']
    (ee1, eb1), (ee2, eb2) = params['enc_edge']
    (w1, b1), (w2, b2) = params['edge_model']
    (m1, mb1), (m2, mb2) = params['msg']
    ((wu, bu),) = params['update']
    (c1, cb1), (c2, cb2) = params['cls_edge']

    row2 = lambda b: b.reshape(1, -1)
    z8 = jnp.zeros((1, 8), jnp.float32)
    m1x_p = jnp.concatenate([m1[0:64], jnp.zeros((64, 8), jnp.float32)], 1)
    m1e_p = jnp.concatenate([m1[64:80], jnp.zeros((16, 8), jnp.float32)], 1)
    m2_p = jnp.concatenate([m2, jnp.zeros((8, 32), jnp.float32)], 0)
    mb1_p = jnp.concatenate([row2(mb1), z8], 1)
    mc1 = jnp.concatenate([m1e_p, c1], 1)
    mcb = jnp.concatenate([mb1_p, row2(cb1)], 1)
    z64 = jnp.zeros((64, 64), jnp.float32)
    bdx = jnp.block([[m1x_p, z64], [z64, m1x_p]])
    z6432 = jnp.zeros((64, 32), jnp.float32)
    bdm = jnp.block([[m2_p, z6432], [z6432, m2_p]])
    bmb2 = jnp.concatenate([row2(mb2), row2(mb2)], 1)
    ew = (w1, row2(b1), w2, row2(b2),
          mc1, mcb, bdx, bdm, bmb2, c2, row2(cb2))

    x0, xin = _node_enc(x, we1, row2(be1), we2, row2(be2))
    e0 = _edge_enc(attr_p, ee1, row2(eb1), ee2, row2(eb2))
    el = e0

    outs = []
    first_class_step = NUM_ENC_STEPS - NUM_CLASS_STEPS + 1
    for step in range(1, NUM_ENC_STEPS + 1):
        xr, xc = _gather2(xin, row_g, col_g)
        el, mcol, mrow, logit = _edge_step(xr, xc, e0, el, ew)
        agg0 = jnp.zeros((N_PAD, 32), jnp.float32).at[col_s].add(mcol)
        agg1 = jnp.zeros((N_PAD, 32), jnp.float32).at[row_s].add(mrow)
        agg2 = jnp.stack([agg0, agg1], axis=0)
        xin = _update(agg2, x0, wu, bu)
        if step >= first_class_step:
            outs.append(logit[:N_EDGES])
    return jnp.stack(outs, axis=0)


# speedup vs baseline: 1.5266x; 1.0388x over previous
import functools

import jax
import jax.numpy as jnp
from jax import lax
from jax.experimental import pallas as pl
from jax.experimental.pallas import tpu as pltpu
from jax.experimental.pallas import tpu_sc as plsc

N_NODES = 10000
N_EDGES = 320000
NUM_ENC_STEPS = 12
NUM_CLASS_STEPS = 11

NC = 2
NS = 16
NW = NC * NS
CH = 128
K = 80
E_PAD = NW * K * CH
BE = 4096
N_PAD = 10240


def _relu(v):
    return jnp.maximum(v, 0.0)


def _node_enc_body(x_ref, w1_ref, b1_ref, w2_ref, b2_ref, x0_ref, xin_ref):
    h = _relu(jnp.dot(x_ref[...], w1_ref[...],
                      preferred_element_type=jnp.float32) + b1_ref[...])
    x0 = _relu(jnp.dot(h, w2_ref[...],
                       preferred_element_type=jnp.float32) + b2_ref[...])
    x0_ref[...] = x0
    xin_ref[:, 0:32] = x0
    xin_ref[:, 32:64] = x0
    xin_ref[:, 64:128] = jnp.zeros((N_NODES, 64), jnp.float32)


def _node_enc(x, w1, b1, w2, b2):
    return pl.pallas_call(
        _node_enc_body,
        out_shape=(jax.ShapeDtypeStruct((N_NODES, 32), jnp.float32),
                   jax.ShapeDtypeStruct((N_NODES, 128), jnp.float32)),
    )(x, w1, b1, w2, b2)


def _edge_enc_body(a_ref, w1_ref, b1_ref, w2_ref, b2_ref, e0_ref):
    h = _relu(jnp.dot(a_ref[...], w1_ref[...],
                      preferred_element_type=jnp.float32) + b1_ref[...])
    e0_ref[...] = _relu(jnp.dot(h, w2_ref[...],
                                preferred_element_type=jnp.float32) + b2_ref[...])


def _edge_enc(attr, w1, b1, w2, b2):
    nb = E_PAD // BE
    return pl.pallas_call(
        _edge_enc_body,
        grid=(nb,),
        in_specs=[
            pl.BlockSpec((BE, 4), lambda i: (i, 0)),
            pl.BlockSpec((4, 16), lambda i: (0, 0)),
            pl.BlockSpec((1, 16), lambda i: (0, 0)),
            pl.BlockSpec((16, 16), lambda i: (0, 0)),
            pl.BlockSpec((1, 16), lambda i: (0, 0)),
        ],
        out_specs=pl.BlockSpec((BE, 16), lambda i: (i, 0)),
        out_shape=jax.ShapeDtypeStruct((E_PAD, 16), jnp.float32),
    )(attr, w1, b1, w2, b2)


def _edge_step_body(xr_ref, xc_ref, e0_ref, el_ref,
                    w1_ref, b1_ref,
                    w2_ref, b2_ref,
                    mc1_ref, mcb_ref, bdx_ref, bdm_ref, bmb2_ref,
                    c2_ref, cb2_ref,
                    el_out_ref, mcol_ref, mrow_ref, logit_ref):
    xr = xr_ref[:, 0:64]
    xc = xc_ref[:, 0:64]
    f32 = jnp.float32
    cat1 = jnp.concatenate([xr, xc, e0_ref[...], el_ref[...]], axis=1)
    h = _relu(jnp.dot(cat1, w1_ref[...], preferred_element_type=f32)
              + b1_ref[...])
    el_new = _relu(jnp.dot(h, w2_ref[...], preferred_element_type=f32)
                   + b2_ref[...])
    el_out_ref[...] = el_new
    d2 = jnp.dot(el_new, mc1_ref[...], preferred_element_type=f32) + mcb_ref[...]
    ecomp2 = jnp.concatenate([d2[:, 0:64], d2[:, 0:64]], axis=1)
    hcr = _relu(jnp.dot(cat1[:, 0:128], bdx_ref[...],
                        preferred_element_type=f32) + ecomp2)
    mcr = _relu(jnp.dot(hcr, bdm_ref[...], preferred_element_type=f32)
                + bmb2_ref[...])
    mcol_ref[...] = mcr[:, 0:32]
    mrow_ref[...] = mcr[:, 32:64]
    g = _relu(d2[:, 64:72])
    logit_ref[...] = (jnp.dot(g, c2_ref[...], preferred_element_type=f32)
                      + cb2_ref[...])


def _edge_step(xr, xc, e0, el, ew):
    nb = E_PAD // BE
    wspec = lambda r, c: pl.BlockSpec((r, c), lambda i: (0, 0))
    return pl.pallas_call(
        _edge_step_body,
        grid=(nb,),
        in_specs=[
            pl.BlockSpec((BE, 128), lambda i: (i, 0)),
            pl.BlockSpec((BE, 128), lambda i: (i, 0)),
            pl.BlockSpec((BE, 16), lambda i: (i, 0)),
            pl.BlockSpec((BE, 16), lambda i: (i, 0)),
            wspec(160, 80),
            wspec(1, 80), wspec(80, 16), wspec(1, 16),
            wspec(16, 72), wspec(1, 72),
            wspec(128, 128), wspec(128, 64), wspec(1, 64),
            wspec(8, 1), wspec(1, 1),
        ],
        out_specs=(
            pl.BlockSpec((BE, 16), lambda i: (i, 0)),
            pl.BlockSpec((BE, 32), lambda i: (i, 0)),
            pl.BlockSpec((BE, 32), lambda i: (i, 0)),
            pl.BlockSpec((BE, 1), lambda i: (i, 0)),
        ),
        out_shape=(
            jax.ShapeDtypeStruct((E_PAD, 16), jnp.float32),
            jax.ShapeDtypeStruct((E_PAD, 32), jnp.float32),
            jax.ShapeDtypeStruct((E_PAD, 32), jnp.float32),
            jax.ShapeDtypeStruct((E_PAD, 1), jnp.float32),
        ),
    )(xr, xc, e0, el, *ew)


def _update_body(agg_ref, x0_ref, wu_ref, bu_ref, xin_ref):
    agg = agg_ref[0, :N_NODES, :] + agg_ref[1, :N_NODES, :]
    xl = _relu(jnp.dot(agg, wu_ref[...], preferred_element_type=jnp.float32)
               + bu_ref[...])
    xin_ref[:, 0:32] = x0_ref[...]
    xin_ref[:, 32:64] = xl
    xin_ref[:, 64:128] = jnp.zeros((N_NODES, 64), jnp.float32)


def _update(agg2, x0, wu, bu):
    return pl.pallas_call(
        _update_body,
        out_shape=jax.ShapeDtypeStruct((N_NODES, 128), jnp.float32),
    )(agg2, x0, wu, bu)


@functools.cache
def _sc_mesh():
    return plsc.VectorSubcoreMesh(core_axis_name="c", subcore_axis_name="s",
                                  num_cores=NC, num_subcores=NS)


def _sc_gather_body(xin_hbm, rowg_hbm, colg_hbm, xr_hbm, xc_hbm,
                    idx_v, buf0, buf1, buf2, buf3,
                    g0, g1, g2, g3, w0, w1, w2, w3):
    c = lax.axis_index("c")
    s = lax.axis_index("s")
    w = s * NC + c
    base = w * (K * CH)
    bufs = (buf0, buf1, buf2, buf3)
    gsems = (g0, g1, g2, g3)
    wsems = (w0, w1, w2, w3)
    nb = 4
    for idx_hbm, out_hbm in ((rowg_hbm, xr_hbm), (colg_hbm, xc_hbm)):
        pltpu.sync_copy(idx_hbm.at[w], idx_v)

        def step(jj, carry):
            j0 = nb * jj
            gds = [pltpu.async_copy(xin_hbm.at[idx_v.at[j0 + b]],
                                    bufs[b], gsems[b]) for b in range(nb)]
            for b in range(nb):
                gds[b].wait()
            wds = [pltpu.async_copy(
                bufs[b], out_hbm.at[pl.ds(base + (j0 + b) * CH, CH)],
                wsems[b]) for b in range(nb)]
            for b in range(nb):
                wds[b].wait()
            return carry

        lax.fori_loop(0, K // nb, step, 0)


def _gather2(xin, rowg3, colg3):
    return pl.kernel(
        _sc_gather_body,
        out_type=(jax.ShapeDtypeStruct((E_PAD, 128), jnp.float32),
                  jax.ShapeDtypeStruct((E_PAD, 128), jnp.float32)),
        mesh=_sc_mesh(),
        scratch_types=[pltpu.VMEM((K, CH), jnp.int32)]
        + [pltpu.VMEM((CH, 128), jnp.float32)] * 4
        + [pltpu.SemaphoreType.DMA] * 8,
    )(xin, rowg3, colg3)


def kernel(x, edge_index, edge_attr, params):
    row, col = edge_index[0], edge_index[1]
    pad = E_PAD - N_EDGES
    row_g = jnp.concatenate([row, jnp.zeros((pad,), jnp.int32)]).reshape(
        NW, K, CH)
    col_g = jnp.concatenate([col, jnp.zeros((pad,), jnp.int32)]).reshape(
        NW, K, CH)
    row_s = jnp.concatenate([row, jnp.full((pad,), N_NODES, jnp.int32)])
    col_s = jnp.concatenate([col, jnp.full((pad,), N_NODES, jnp.int32)])
    attr_p = jnp.concatenate(
        [edge_attr, jnp.zeros((pad, 4), jnp.float32)], axis=0)

    (we1, be1), (we2, be2) = params['enc_node']
    (ee1, eb1), (ee2, eb2) = params['enc_edge']
    (w1, b1), (w2, b2) = params['edge_model']
    (m1, mb1), (m2, mb2) = params['msg']
    ((wu, bu),) = params['update']
    (c1, cb1), (c2, cb2) = params['cls_edge']

    row2 = lambda b: b.reshape(1, -1)
    z8 = jnp.zeros((1, 8), jnp.float32)
    m1x_p = jnp.concatenate([m1[0:64], jnp.zeros((64, 8), jnp.float32)], 1)
    m1e_p = jnp.concatenate([m1[64:80], jnp.zeros((16, 8), jnp.float32)], 1)
    m2_p = jnp.concatenate([m2, jnp.zeros((8, 32), jnp.float32)], 0)
    mb1_p = jnp.concatenate([row2(mb1), z8], 1)
    mc1 = jnp.concatenate([m1e_p, c1], 1)
    mcb = jnp.concatenate([mb1_p, row2(cb1)], 1)
    z64 = jnp.zeros((64, 64), jnp.float32)
    bdx = jnp.block([[m1x_p, z64], [z64, m1x_p]])
    z6432 = jnp.zeros((64, 32), jnp.float32)
    bdm = jnp.block([[m2_p, z6432], [z6432, m2_p]])
    bmb2 = jnp.concatenate([row2(mb2), row2(mb2)], 1)
    ew = (w1, row2(b1), w2, row2(b2),
          mc1, mcb, bdx, bdm, bmb2, c2, row2(cb2))

    x0, xin = _node_enc(x, we1, row2(be1), we2, row2(be2))
    e0 = _edge_enc(attr_p, ee1, row2(eb1), ee2, row2(eb2))
    el = e0

    outs = []
    first_class_step = NUM_ENC_STEPS - NUM_CLASS_STEPS + 1
    for step in range(1, NUM_ENC_STEPS + 1):
        xr, xc = _gather2(xin, row_g, col_g)
        el, mcol, mrow, logit = _edge_step(xr, xc, e0, el, ew)
        agg0 = jnp.zeros((N_PAD, 32), jnp.float32).at[col_s].add(mcol)
        agg1 = jnp.zeros((N_PAD, 32), jnp.float32).at[row_s].add(mrow)
        agg2 = jnp.stack([agg0, agg1], axis=0)
        xin = _update(agg2, x0, wu, bu)
        if step >= first_class_step:
            outs.append(logit[:N_EDGES])
    return jnp.stack(outs, axis=0)
